# baseline (speedup 1.0000x reference)
import jax
import jax.numpy as jnp
from jax.experimental import pallas as pl

B0, B1, C, NB = 4096, 512, 4, 16
BB = 32


def _body(x_ref, o_ref):
    x = x_ref[...]
    z_lo = jnp.zeros((BB, B1, 1), x.dtype)
    z_hi = jnp.zeros((BB, B1, NB - C - 1), x.dtype)
    o_ref[...] = jnp.concatenate([z_lo, x, z_hi], axis=-1)


def kernel(inputs):
    return pl.pallas_call(
        _body,
        grid=(B0 // BB,),
        in_specs=[pl.BlockSpec((BB, B1, C), lambda i: (i, 0, 0))],
        out_specs=pl.BlockSpec((BB, B1, NB), lambda i: (i, 0, 0)),
        out_shape=jax.ShapeDtypeStruct((B0, B1, NB), jnp.float32),
    )(inputs)
